# baseline (speedup 1.0000x reference)
import functools

import jax
import jax.numpy as jnp
from jax import lax
from jax.experimental import pallas as pl
from jax.experimental.pallas import tpu as pltpu
from jax.experimental.pallas import tpu_sc as plsc

N = 10000
NPAD = 10240
E = 320000
DPAD = 64

NC = 2
NS = 16
E_PER_TILE = E // (NC * NS)
N_PER_TILE = NPAD // NS

K1 = 200
K2 = 1000
K3 = 400


def _tc_a_body(x_ref, wl_ref, wr_ref, b_ref, y1aug_ref, hr_ref):
    xb = x_ref[...]
    y1 = jnp.dot(xb, wl_ref[...], preferred_element_type=jnp.float32)
    r = xb.shape[0]
    y1aug_ref[...] = y1
    hr_ref[...] = jnp.dot(xb, wr_ref[...], preferred_element_type=jnp.float32) + b_ref[...]


def _tc_a(x, wl1, wr1, b1):
    blk = 1000
    grid = N // blk
    return pl.pallas_call(
        _tc_a_body,
        grid=(grid,),
        in_specs=[
            pl.BlockSpec((blk, 128), lambda i: (i, 0)),
            pl.BlockSpec((128, 64), lambda i: (0, 0)),
            pl.BlockSpec((128, 64), lambda i: (0, 0)),
            pl.BlockSpec((1, 64), lambda i: (0, 0)),
        ],
        out_specs=[
            pl.BlockSpec((blk, DPAD), lambda i: (i, 0)),
            pl.BlockSpec((blk, 64), lambda i: (i, 0)),
        ],
        out_shape=[
            jax.ShapeDtypeStruct((N, DPAD), jnp.float32),
            jax.ShapeDtypeStruct((N, 64), jnp.float32),
        ],
    )(x, wl1, wr1, b1)


def _tc_b_body(acc_ref, cnt_ref, hr_ref, wl2_ref, wr2_ref, b2_ref, wf2t_ref,
               wf1t_ref, y2_ref, zr_ref, inv_ref, wrow_ref):
    s = acc_ref[0] + acc_ref[1]
    cnt = cnt_ref[0] + cnt_ref[1]
    inv = 1.0 / jnp.maximum(cnt, 1.0)
    h = s * inv + hr_ref[...]
    y2_ref[...] = jnp.dot(h, wl2_ref[...], preferred_element_type=jnp.float32)
    zr_ref[...] = jnp.dot(h, wr2_ref[...], preferred_element_type=jnp.float32) + b2_ref[...]
    inv_ref[...] = inv
    wrow_ref[...] = jnp.dot(wf2t_ref[...], wf1t_ref[...], preferred_element_type=jnp.float32)


def _tc_b(acc1, cnt, hr, wl2, wr2, b2, wf2t, wf1t):
    blk = 1000
    grid = N // blk
    return pl.pallas_call(
        _tc_b_body,
        grid=(grid,),
        in_specs=[
            pl.BlockSpec((2, blk, DPAD), lambda i: (0, i, 0)),
            pl.BlockSpec((2, blk, 1), lambda i: (0, i, 0)),
            pl.BlockSpec((blk, 64), lambda i: (i, 0)),
            pl.BlockSpec((64, 16), lambda i: (0, 0)),
            pl.BlockSpec((64, 16), lambda i: (0, 0)),
            pl.BlockSpec((1, 16), lambda i: (0, 0)),
            pl.BlockSpec((1, 8), lambda i: (0, 0)),
            pl.BlockSpec((8, 16), lambda i: (0, 0)),
        ],
        out_specs=[
            pl.BlockSpec((blk, 16), lambda i: (i, 0)),
            pl.BlockSpec((blk, 16), lambda i: (i, 0)),
            pl.BlockSpec((blk, 1), lambda i: (i, 0)),
            pl.BlockSpec((1, 16), lambda i: (0, 0)),
        ],
        out_shape=[
            jax.ShapeDtypeStruct((N, 16), jnp.float32),
            jax.ShapeDtypeStruct((NPAD, 16), jnp.float32),
            jax.ShapeDtypeStruct((NPAD, 1), jnp.float32),
            jax.ShapeDtypeStruct((1, 16), jnp.float32),
        ],
    )(acc1, cnt, hr, wl2, wr2, b2, wf2t, wf1t)


def _make_sc_agg(dcols, kchunk, with_cnt):
    nchunks = E_PER_TILE // kchunk
    NB = 4
    NI = 6
    mesh = plsc.VectorSubcoreMesh(core_axis_name="c", subcore_axis_name="s")

    scratch = (
        [pltpu.VMEM((kchunk,), jnp.int32) for _ in range(NI)]
        + [pltpu.VMEM((kchunk,), jnp.int32) for _ in range(NI)]
        + [pltpu.VMEM((kchunk, dcols), jnp.float32) for _ in range(NB)]
        + [pltpu.VMEM_SHARED((NPAD, dcols), jnp.float32)]
        + [pltpu.SemaphoreType.DMA for _ in range(NI + 2 * NB)]
    )
    out_type = [jax.ShapeDtypeStruct((NC, NPAD, dcols), jnp.float32)]
    if with_cnt:
        scratch += (
            [pltpu.VMEM(((kchunk + 15) // 16 * 16,), jnp.float32)]
            + [pltpu.VMEM_SHARED((NPAD,), jnp.float32)]
            + [pltpu.SemaphoreType.DMA for _ in range(NB)]
        )
        out_type.append(jax.ShapeDtypeStruct((NC, NPAD), jnp.float32))

    @functools.partial(
        pl.kernel,
        mesh=mesh,
        compiler_params=pltpu.CompilerParams(use_tc_tiling_on_sc=False),
        out_type=out_type,
        scratch_types=scratch,
    )
    def agg(table, srcs, dsts, zrows, zc, *outs_sc):
        if with_cnt:
            acc_out, cnt_out = outs_sc[0], outs_sc[1]
            sc = outs_sc[2:]
        else:
            acc_out = outs_sc[0]
            sc = outs_sc[1:]
        sidx = sc[0:NI]
        didx = sc[NI:2 * NI]
        rows = sc[2 * NI:2 * NI + NB]
        acc_sh = sc[2 * NI + NB]
        o = 2 * NI + NB + 1
        sem_i = sc[o:o + NI]
        sem_g = sc[o + NI:o + NI + NB]
        sem_s = sc[o + NI + NB:o + NI + 2 * NB]
        if with_cnt:
            o2 = o + NI + 2 * NB
            ones_v = sc[o2]
            cnt_sh = sc[o2 + 1]
            sem_c = sc[o2 + 2:o2 + 2 + NB]

        c = lax.axis_index("c")
        s = lax.axis_index("s")
        pltpu.sync_copy(zrows, acc_sh.at[pl.ds(s * N_PER_TILE, N_PER_TILE)])
        if with_cnt:
            def fill(i, carry):
                ones_v[pl.ds(i * 16, 16)] = jnp.ones((16,), jnp.float32)
                return carry
            lax.fori_loop(0, (kchunk + 15) // 16, fill, 0)
            pltpu.sync_copy(zc, cnt_sh.at[pl.ds(s * N_PER_TILE, N_PER_TILE)])
        plsc.subcore_barrier()
        base = c * (E // NC) + s * E_PER_TILE

        idx_d = {}
        gat_d = {}
        sca_d = {}
        cnt_d = {}

        def start_idx(j):
            b = j % NI
            off = base + j * kchunk
            idx_d[j] = (
                pltpu.async_copy(srcs.at[pl.ds(off, kchunk)], sidx[b], sem_i[b]),
                pltpu.async_copy(dsts.at[pl.ds(off, kchunk)], didx[b], sem_i[b]),
            )

        def start_gather(j):
            gat_d[j] = pltpu.async_copy(
                table.at[sidx[j % NI]], rows[j % NB], sem_g[j % NB])

        def start_scatter(j):
            sca_d[j] = pltpu.async_copy(
                rows[j % NB], acc_sh.at[didx[j % NI]], sem_s[j % NB], add=True)
            if with_cnt:
                cnt_d[j] = pltpu.async_copy(
                    ones_v.at[pl.ds(0, kchunk)], cnt_sh.at[didx[j % NI]],
                    sem_c[j % NB], add=True)

        start_idx(0)
        if nchunks > 1:
            start_idx(1)
        idx_d[0][0].wait()
        idx_d[0][1].wait()
        start_gather(0)
        for j in range(nchunks):
            gat_d[j].wait()
            if j + 1 < nchunks:
                idx_d[j + 1][0].wait()
                idx_d[j + 1][1].wait()
                if j + 1 >= NB:
                    sca_d[j + 1 - NB].wait()
                    if with_cnt:
                        cnt_d[j + 1 - NB].wait()
                start_gather(j + 1)
            start_scatter(j)
            if j + 2 < nchunks:
                start_idx(j + 2)
        for j in range(max(0, nchunks - NB), nchunks):
            sca_d[j].wait()
            if with_cnt:
                cnt_d[j].wait()

        plsc.subcore_barrier()
        pltpu.sync_copy(
            acc_sh.at[pl.ds(s * N_PER_TILE, N_PER_TILE)],
            acc_out.at[c, pl.ds(s * N_PER_TILE, N_PER_TILE)],
        )
        if with_cnt:
            pltpu.sync_copy(
                cnt_sh.at[pl.ds(s * N_PER_TILE, N_PER_TILE)],
                cnt_out.at[c, pl.ds(s * N_PER_TILE, N_PER_TILE)],
            )

    return agg


_sc_agg1 = _make_sc_agg(DPAD, K1, with_cnt=True)


def _make_sc_tail():
    EA = E // NS
    nchA = EA // K2
    NBa = 2
    NIa = 4
    SB = 320
    nchC = E_PER_TILE // K3
    ngroups = K3 // 16
    NIc = 3
    mesh = plsc.VectorSubcoreMesh(core_axis_name="c", subcore_axis_name="s")

    scratch = (
        [pltpu.VMEM((K2,), jnp.int32) for _ in range(NIa)]
        + [pltpu.VMEM((K2,), jnp.int32) for _ in range(NIa)]
        + [pltpu.VMEM((K2, 16), jnp.float32) for _ in range(NBa)]
        + [pltpu.VMEM_SHARED((NPAD, 16), jnp.float32)]
        + [pltpu.SemaphoreType.DMA for _ in range(NIa + 2 * NBa)]
        + [pltpu.VMEM((SB, 16), jnp.float32) for _ in range(4)]
        + [pltpu.VMEM((SB,), jnp.float32)]
        + [pltpu.VMEM((16,), jnp.float32)]
        + [pltpu.VMEM((K3,), jnp.int32) for _ in range(NIc)]
        + [pltpu.VMEM((K3,), jnp.int32) for _ in range(NIc)]
        + [pltpu.VMEM((K3, 16), jnp.float32) for _ in range(2)]
        + [pltpu.VMEM((K3, 16), jnp.float32) for _ in range(2)]
        + [pltpu.VMEM((K3,), jnp.float32) for _ in range(2)]
        + [pltpu.VMEM((16,), jnp.float32)]
        + [pltpu.SemaphoreType.DMA for _ in range(NIc + 6)]
    )

    @functools.partial(
        pl.kernel,
        mesh=mesh,
        compiler_params=pltpu.CompilerParams(
            use_tc_tiling_on_sc=False, needs_layout_passes=False),
        out_type=[
            jax.ShapeDtypeStruct((E,), jnp.float32),
            jax.ShapeDtypeStruct((NPAD, 16), jnp.float32),
            jax.ShapeDtypeStruct((NPAD, 16), jnp.float32),
        ],
        scratch_types=scratch,
    )
    def tail(y2, srcs, dsts, invf, zrp, wrow, cvec, zrows, out, z_hbm, zw_hbm, *sc):
        o = 0
        a_sidx = sc[o:o + NIa]; o += NIa
        a_didx = sc[o:o + NIa]; o += NIa
        a_rows = sc[o:o + NBa]; o += NBa
        acc_sh = sc[o]; o += 1
        a_sem_i = sc[o:o + NIa]; o += NIa
        a_sem_g = sc[o:o + NBa]; o += NBa
        a_sem_s = sc[o:o + NBa]; o += NBa
        s2_loc, zr_loc, zv_loc, zwv_loc = sc[o:o + 4]; o += 4
        inv_loc = sc[o]; o += 1
        wv = sc[o]; o += 1
        c_sidx = sc[o:o + NIc]; o += NIc
        c_didx = sc[o:o + NIc]; o += NIc
        zs = sc[o:o + 2]; o += 2
        zd = sc[o:o + 2]; o += 2
        ov = sc[o:o + 2]; o += 2
        cv = sc[o]; o += 1
        sem_i = sc[o:o + NIc]; o += NIc
        sem_zs = sc[o:o + 2]; o += 2
        sem_zd = sc[o:o + 2]; o += 2
        sem_o = sc[o:o + 2]; o += 2

        c = lax.axis_index("c")
        s = lax.axis_index("s")
        pltpu.sync_copy(zrows, acc_sh.at[pl.ds(s * N_PER_TILE, N_PER_TILE)])
        pltpu.sync_copy(cvec, cv)
        pltpu.sync_copy(wrow, wv)
        plsc.subcore_barrier()

        base_a = s * EA
        idx_a = {}
        gat_a = {}
        sca_a = {}

        def a_start_idx(j):
            b = j % NIa
            off = base_a + j * K2
            idx_a[j] = (
                pltpu.async_copy(srcs.at[pl.ds(off, K2)], a_sidx[b], a_sem_i[b]),
                pltpu.async_copy(dsts.at[pl.ds(off, K2)], a_didx[b], a_sem_i[b]),
            )

        def a_start_gather(j):
            gat_a[j] = pltpu.async_copy(
                y2.at[a_sidx[j % NIa]], a_rows[j % NBa], a_sem_g[j % NBa])

        def a_start_scatter(j):
            sca_a[j] = pltpu.async_copy(
                a_rows[j % NBa], acc_sh.at[a_didx[j % NIa]], a_sem_s[j % NBa],
                add=True)

        a_start_idx(0)
        a_start_idx(1)
        idx_a[0][0].wait()
        idx_a[0][1].wait()
        a_start_gather(0)
        for j in range(nchA):
            gat_a[j].wait()
            if j + 1 < nchA:
                idx_a[j + 1][0].wait()
                idx_a[j + 1][1].wait()
                if j + 1 >= NBa:
                    sca_a[j + 1 - NBa].wait()
                a_start_gather(j + 1)
            a_start_scatter(j)
            if j + 2 < nchA:
                a_start_idx(j + 2)
        for j in range(max(0, nchA - NBa), nchA):
            sca_a[j].wait()
        plsc.subcore_barrier()

        wval = wv[...]
        for t in range(N_PER_TILE // SB):
            r0 = s * N_PER_TILE + t * SB
            pltpu.sync_copy(acc_sh.at[pl.ds(r0, SB)], s2_loc)
            pltpu.sync_copy(zrp.at[pl.ds(r0, SB)], zr_loc)
            pltpu.sync_copy(invf.at[pl.ds(r0, SB)], inv_loc)

            def bgroup(i, carry):
                ivec = inv_loc[pl.ds(i * 16, 16)]
                for k in range(16):
                    r = i * 16 + k
                    zrow = s2_loc[r] * ivec[k] + zr_loc[r]
                    zv_loc[r] = zrow
                    zwv_loc[r] = zrow * wval
                return carry

            lax.fori_loop(0, SB // 16, bgroup, 0)
            pltpu.sync_copy(zv_loc, z_hbm.at[pl.ds(r0, SB)])
            pltpu.sync_copy(zwv_loc, zw_hbm.at[pl.ds(r0, SB)])
        plsc.subcore_barrier()

        base = c * (E // NC) + s * E_PER_TILE
        cval = cv[...]
        zval = jnp.zeros((16,), jnp.float32)
        lanes = lax.iota(jnp.int32, 16)

        idx_d = {}
        gat_d = {}
        out_d = {}

        def start_idx(j):
            b = j % NIc
            off = base + j * K3
            idx_d[j] = (
                pltpu.async_copy(srcs.at[pl.ds(off, K3)], c_sidx[b], sem_i[b]),
                pltpu.async_copy(dsts.at[pl.ds(off, K3)], c_didx[b], sem_i[b]),
            )

        def start_gathers(j):
            b = j % 2
            gat_d[j] = (
                pltpu.async_copy(z_hbm.at[c_sidx[j % NIc]], zs[b], sem_zs[b]),
                pltpu.async_copy(zw_hbm.at[c_didx[j % NIc]], zd[b], sem_zd[b]),
            )

        start_idx(0)
        start_idx(1)
        idx_d[0][0].wait()
        idx_d[0][1].wait()
        start_gathers(0)
        for j in range(nchC):
            b = j % 2
            gat_d[j][0].wait()
            gat_d[j][1].wait()
            if j + 1 < nchC:
                idx_d[j + 1][0].wait()
                idx_d[j + 1][1].wait()
                start_gathers(j + 1)
            if j + 2 < nchC:
                start_idx(j + 2)
            if j >= 2:
                out_d[j - 2].wait()

            zsb = zs[b]
            zdb = zd[b]
            ovb = ov[b]

            def group(i, carry2):
                rows = i * 16 + lanes
                parts = [cval, zval, zval, zval]
                for d in range(16):
                    cols = (lanes + d) & 15
                    sv = plsc.load_gather(zsb, [rows, cols])
                    dv = plsc.load_gather(zdb, [rows, cols])
                    parts[d % 4] = parts[d % 4] + sv * dv
                acc = (parts[0] + parts[1]) + (parts[2] + parts[3])
                sig = 1.0 / (1.0 + jnp.exp(-acc))
                ovb[pl.ds(i * 16, 16)] = sig
                return carry2

            lax.fori_loop(0, ngroups, group, 0)
            out_d[j] = pltpu.async_copy(
                ovb, out.at[pl.ds(base + j * K3, K3)], sem_o[b])
        for j in range(max(0, nchC - 2), nchC):
            out_d[j].wait()

    return tail


_sc_tail = _make_sc_tail()


def kernel(x, edge_index, Wl1, Wr1, b1, Wl2, Wr2, b2, W_fc1, b_fc1, W_fc2, b_fc2):
    src = edge_index[0]
    dst = edge_index[1]

    y1aug, hr = _tc_a(x, Wl1, Wr1, b1.reshape(1, 64))
    zrows1 = jnp.zeros((N_PER_TILE, DPAD), jnp.float32)
    zc = jnp.zeros((N_PER_TILE,), jnp.float32)
    acc1, cnt = _sc_agg1(y1aug, src, dst, zrows1, zc)

    y2, zrp, invp, wrow = _tc_b(acc1, cnt.reshape(NC, NPAD, 1), hr, Wl2, Wr2,
                                b2.reshape(1, 16), W_fc2.T, W_fc1.T)

    cscalar = jnp.dot(b_fc1, W_fc2[:, 0]) + b_fc2[0]
    cvec = jnp.full((16,), cscalar, jnp.float32)
    zrows2 = jnp.zeros((N_PER_TILE, 16), jnp.float32)
    out, _, _ = _sc_tail(y2, src, dst, invp.reshape(NPAD), zrp,
                         wrow.reshape(16), cvec, zrows2)
    return out.reshape(E, 1)
